# baseline (speedup 1.0000x reference)
import functools

import jax
import jax.numpy as jnp
from jax import lax
from jax.experimental import pallas as pl
from jax.experimental.pallas import tpu as pltpu
from jax.experimental.pallas import tpu_sc as plsc


def _stage1_body(f_ref, mem_ref, gc_ref, fg_ref, score_ref):
    f = f_ref[...]
    D = f.shape[0]
    C = f.shape[2]
    ones = jnp.ones((C,), jnp.float32)
    colmean = lax.dot_general(f, ones, (((2,), (0,)), ((), ())),
                              preferred_element_type=jnp.float32) * (1.0 / C)
    a = colmean - jnp.max(colmean, axis=0, keepdims=True)
    e = jnp.exp(a)
    se = jnp.sum(e, axis=0)
    fg = lax.dot_general(e, f, (((0,), (0,)), ((1,), (1,))),
                         preferred_element_type=jnp.float32)
    fg = fg * ((1.0 / D) / se[:, None])
    fg_ref[...] = fg
    score = lax.dot_general(fg, mem_ref[...], (((1,), (1,)), ((), ())),
                            preferred_element_type=jnp.float32)
    score_ref[...] = score
    rmax = jnp.max(score, axis=1, keepdims=True)
    es = jnp.exp(score - rmax)
    resp = lax.dot_general(es, mem_ref[...], (((1,), (0,)), ((), ())),
                           preferred_element_type=jnp.float32)
    mr = fg + resp / jnp.sum(es, axis=1, keepdims=True)
    gc_ref[...] = f + mr[None, :, :]


def _stage1(ft, memory, bb):
    D, B, C = ft.shape
    M = memory.shape[0]
    nb = B // bb
    return pl.pallas_call(
        _stage1_body,
        grid=(nb,),
        in_specs=[
            pl.BlockSpec((D, bb, C), lambda i: (0, i, 0)),
            pl.BlockSpec((M, C), lambda i: (0, 0)),
        ],
        out_specs=[
            pl.BlockSpec((D, bb, C), lambda i: (0, i, 0)),
            pl.BlockSpec((bb, C), lambda i: (i, 0)),
            pl.BlockSpec((bb, M), lambda i: (i, 0)),
        ],
        out_shape=[
            jax.ShapeDtypeStruct((D, B, C), jnp.float32),
            jax.ShapeDtypeStruct((B, C), jnp.float32),
            jax.ShapeDtypeStruct((B, M), jnp.float32),
        ],
    )(ft, memory)


def _stage15_body(score_ref, cmax_ref, csum_ref, idx_ref, rmax_ref):
    score = score_ref[...]
    M = score.shape[1]
    cmax = jnp.max(score, axis=0, keepdims=True)
    csum = jnp.sum(jnp.exp(score - cmax), axis=0, keepdims=True)
    cmax_ref[...] = cmax
    csum_ref[...] = csum
    rmax = jnp.max(score, axis=1, keepdims=True)
    ii = lax.broadcasted_iota(jnp.int32, score.shape, 1)
    idxv = jnp.min(jnp.where(score == rmax, ii, M), axis=1)
    idx_ref[...] = idxv[None, :]
    rmax_ref[...] = rmax[:, 0][None, :]


def _stage15(score):
    B, M = score.shape
    return pl.pallas_call(
        _stage15_body,
        out_shape=[
            jax.ShapeDtypeStruct((1, M), jnp.float32),
            jax.ShapeDtypeStruct((1, M), jnp.float32),
            jax.ShapeDtypeStruct((1, B), jnp.int32),
            jax.ShapeDtypeStruct((1, B), jnp.float32),
        ],
    )(score)


_NC, _NS = 2, 16
_NW = _NC * _NS


def _sc_body(fg_hbm, idx_hbm, rmax_hbm, cmax_hbm, csum_hbm, maskf_hbm,
             z_hbm, inc_hbm, idx_v, rm_v, mk_v, cm_v, cs_v, w_v, rows_v,
             shared):
    C = fg_hbm.shape[1]
    bpw = fg_hbm.shape[0] // _NW
    mps = cmax_hbm.shape[0] // _NS
    cid = lax.axis_index("c")
    sid = lax.axis_index("s")
    wid = sid * _NC + cid
    base = wid * bpw

    pltpu.sync_copy(z_hbm.at[pl.ds(sid * mps, mps)],
                    shared.at[pl.ds(sid * mps, mps)])

    pltpu.sync_copy(idx_hbm.at[pl.ds(base, bpw)], idx_v)
    pltpu.sync_copy(rmax_hbm.at[pl.ds(base, bpw)], rm_v)
    pltpu.sync_copy(maskf_hbm.at[pl.ds(base, bpw)], mk_v)
    pltpu.sync_copy(cmax_hbm, cm_v)
    pltpu.sync_copy(csum_hbm, cs_v)
    pltpu.sync_copy(fg_hbm.at[pl.ds(base, bpw)], rows_v)

    for j in range(bpw // 16):
        sl = pl.ds(j * 16, 16)
        iv = idx_v[sl]
        cmx = plsc.load_gather(cm_v, [iv])
        csm = plsc.load_gather(cs_v, [iv])
        w_v[sl] = jnp.exp(rm_v[sl] - cmx) / csm * mk_v[sl]

    def _scale_row(r, _):
        spl = plsc.load_gather(w_v, [jnp.zeros((16,), jnp.int32) + r])
        for ci in range(C // 16):
            sl = pl.ds(ci * 16, 16)
            rows_v[r, sl] = rows_v[r, sl] * spl
        return 0

    lax.fori_loop(0, bpw, _scale_row, 0)
    plsc.subcore_barrier()

    pltpu.sync_copy(rows_v, shared.at[idx_v], add=True)
    plsc.subcore_barrier()

    pltpu.sync_copy(shared.at[pl.ds(sid * mps, mps)],
                    inc_hbm.at[cid, pl.ds(sid * mps, mps)])


def _stage2_sc(fg, idx, rmax, cmax, csum, maskf):
    B, C = fg.shape
    M = cmax.shape[0]
    bpw = B // _NW
    mps = M // _NS
    mesh = plsc.VectorSubcoreMesh(core_axis_name="c", subcore_axis_name="s")
    kern = functools.partial(
        pl.kernel,
        out_type=jax.ShapeDtypeStruct((_NC, M, C), jnp.float32),
        mesh=mesh,
        compiler_params=pltpu.CompilerParams(use_tc_tiling_on_sc=False,
                                             needs_layout_passes=False),
        scratch_types=[
            pltpu.VMEM((bpw,), jnp.int32),
            pltpu.VMEM((bpw,), jnp.float32),
            pltpu.VMEM((bpw,), jnp.float32),
            pltpu.VMEM((M,), jnp.float32),
            pltpu.VMEM((M,), jnp.float32),
            pltpu.VMEM((bpw,), jnp.float32),
            pltpu.VMEM((bpw, C), jnp.float32),
            pltpu.VMEM_SHARED((M, C), jnp.float32),
        ],
    )(_sc_body)
    zeros = jnp.zeros((M, C), jnp.float32)
    return kern(fg, idx, rmax, cmax, csum, maskf, zeros)


def _stage3_body(mem_ref, inc_ref, out_ref):
    um = mem_ref[...] + inc_ref[0] + inc_ref[1]
    nrm = jnp.sqrt(jnp.sum(um * um, axis=1, keepdims=True))
    out_ref[...] = um / jnp.maximum(nrm, 1e-12)


def _stage3(memory, inc):
    M, C = memory.shape
    return pl.pallas_call(
        _stage3_body,
        out_shape=jax.ShapeDtypeStruct((M, C), jnp.float32),
    )(memory, inc)


def kernel(feature, memory, train, mask):
    B, C, D = feature.shape
    M = memory.shape[0]
    ft = jnp.transpose(feature, (2, 0, 1))
    maskf = mask.astype(jnp.float32) * jnp.asarray(train, jnp.float32)
    gct, fg, score = _stage1(ft, memory, 16)
    gc = jnp.transpose(gct, (1, 2, 0))
    cmax, csum, idx2, rmax2 = _stage15(score)
    inc = _stage2_sc(fg, idx2.reshape(B), rmax2.reshape(B),
                     cmax.reshape(M), csum.reshape(M), maskf)
    upd = _stage3(memory, inc)
    return gc, upd


# speedup vs baseline: 1.1643x; 1.1643x over previous
import functools

import jax
import jax.numpy as jnp
from jax import lax
from jax.experimental import pallas as pl
from jax.experimental.pallas import tpu as pltpu
from jax.experimental.pallas import tpu_sc as plsc


def _stage1_body(f_ref, mem_ref, gc_ref, fg_ref, score_ref):
    f = f_ref[...]
    D = f.shape[0]
    C = f.shape[2]
    ones = jnp.ones((C,), jnp.float32)
    colmean = lax.dot_general(f, ones, (((2,), (0,)), ((), ())),
                              preferred_element_type=jnp.float32) * (1.0 / C)
    a = colmean - jnp.max(colmean, axis=0, keepdims=True)
    e = jnp.exp(a)
    se = jnp.sum(e, axis=0)
    fg = lax.dot_general(e, f, (((0,), (0,)), ((1,), (1,))),
                         preferred_element_type=jnp.float32)
    fg = fg * ((1.0 / D) / se[:, None])
    fg_ref[...] = fg
    score = lax.dot_general(fg, mem_ref[...], (((1,), (1,)), ((), ())),
                            preferred_element_type=jnp.float32)
    score_ref[...] = score
    rmax = jnp.max(score, axis=1, keepdims=True)
    es = jnp.exp(score - rmax)
    resp = lax.dot_general(es, mem_ref[...], (((1,), (0,)), ((), ())),
                           preferred_element_type=jnp.float32)
    mr = fg + resp / jnp.sum(es, axis=1, keepdims=True)
    gc_ref[...] = f + mr[None, :, :]


def _stage1(ft, memory, bb):
    D, B, C = ft.shape
    M = memory.shape[0]
    nb = B // bb
    return pl.pallas_call(
        _stage1_body,
        grid=(nb,),
        in_specs=[
            pl.BlockSpec((D, bb, C), lambda i: (0, i, 0)),
            pl.BlockSpec((M, C), lambda i: (0, 0)),
        ],
        out_specs=[
            pl.BlockSpec((D, bb, C), lambda i: (0, i, 0)),
            pl.BlockSpec((bb, C), lambda i: (i, 0)),
            pl.BlockSpec((bb, M), lambda i: (i, 0)),
        ],
        out_shape=[
            jax.ShapeDtypeStruct((D, B, C), jnp.float32),
            jax.ShapeDtypeStruct((B, C), jnp.float32),
            jax.ShapeDtypeStruct((B, M), jnp.float32),
        ],
    )(ft, memory)


def _stage15_body(score_ref, cmax_ref, csum_ref, idx_ref, rmax_ref):
    score = score_ref[...]
    M = score.shape[1]
    cmax = jnp.max(score, axis=0, keepdims=True)
    csum = jnp.sum(jnp.exp(score - cmax), axis=0, keepdims=True)
    cmax_ref[...] = cmax
    csum_ref[...] = csum
    rmax = jnp.max(score, axis=1, keepdims=True)
    ii = lax.broadcasted_iota(jnp.int32, score.shape, 1)
    idxv = jnp.min(jnp.where(score == rmax, ii, M), axis=1)
    idx_ref[...] = idxv[None, :]
    rmax_ref[...] = rmax[:, 0][None, :]


def _stage15(score):
    B, M = score.shape
    return pl.pallas_call(
        _stage15_body,
        out_shape=[
            jax.ShapeDtypeStruct((1, M), jnp.float32),
            jax.ShapeDtypeStruct((1, M), jnp.float32),
            jax.ShapeDtypeStruct((1, B), jnp.int32),
            jax.ShapeDtypeStruct((1, B), jnp.float32),
        ],
    )(score)


_NC, _NS = 2, 16
_NW = _NC * _NS


def _sc_body(fg_hbm, idx_hbm, rmax_hbm, cmax_hbm, csum_hbm, maskf_hbm,
             z_hbm, inc_hbm, idx_v, rm_v, mk_v, cm_v, cs_v, w_v, rows_v,
             shared):
    C = fg_hbm.shape[1]
    bpw = fg_hbm.shape[0] // _NW
    mps = cmax_hbm.shape[0] // _NS
    cid = lax.axis_index("c")
    sid = lax.axis_index("s")
    wid = sid * _NC + cid
    base = wid * bpw

    pltpu.sync_copy(z_hbm.at[pl.ds(sid * mps, mps)],
                    shared.at[pl.ds(sid * mps, mps)])

    pltpu.sync_copy(idx_hbm.at[pl.ds(base, bpw)], idx_v)
    pltpu.sync_copy(rmax_hbm.at[pl.ds(base, bpw)], rm_v)
    pltpu.sync_copy(maskf_hbm.at[pl.ds(base, bpw)], mk_v)
    pltpu.sync_copy(cmax_hbm, cm_v)
    pltpu.sync_copy(csum_hbm, cs_v)
    pltpu.sync_copy(fg_hbm.at[pl.ds(base, bpw)], rows_v)

    for j in range(bpw // 16):
        sl = pl.ds(j * 16, 16)
        iv = idx_v[sl]
        cmx = plsc.load_gather(cm_v, [iv])
        csm = plsc.load_gather(cs_v, [iv])
        w_v[sl] = jnp.exp(rm_v[sl] - cmx) / csm * mk_v[sl]

    def _scale_row(r, _):
        spl = plsc.load_gather(w_v, [jnp.zeros((16,), jnp.int32) + r])
        for ci in range(C // 16):
            sl = pl.ds(ci * 16, 16)
            rows_v[r, sl] = rows_v[r, sl] * spl
        return 0

    lax.fori_loop(0, bpw, _scale_row, 0)
    plsc.subcore_barrier()

    pltpu.sync_copy(rows_v, shared.at[idx_v], add=True)
    plsc.subcore_barrier()

    pltpu.sync_copy(shared.at[pl.ds(sid * mps, mps)],
                    inc_hbm.at[cid, pl.ds(sid * mps, mps)])


def _stage2_sc(fg, idx, rmax, cmax, csum, maskf):
    B, C = fg.shape
    M = cmax.shape[0]
    bpw = B // _NW
    mps = M // _NS
    mesh = plsc.VectorSubcoreMesh(core_axis_name="c", subcore_axis_name="s")
    kern = functools.partial(
        pl.kernel,
        out_type=jax.ShapeDtypeStruct((_NC, M, C), jnp.float32),
        mesh=mesh,
        compiler_params=pltpu.CompilerParams(use_tc_tiling_on_sc=False,
                                             needs_layout_passes=False),
        scratch_types=[
            pltpu.VMEM((bpw,), jnp.int32),
            pltpu.VMEM((bpw,), jnp.float32),
            pltpu.VMEM((bpw,), jnp.float32),
            pltpu.VMEM((M,), jnp.float32),
            pltpu.VMEM((M,), jnp.float32),
            pltpu.VMEM((bpw,), jnp.float32),
            pltpu.VMEM((bpw, C), jnp.float32),
            pltpu.VMEM_SHARED((M, C), jnp.float32),
        ],
    )(_sc_body)
    zeros = jnp.zeros((M, C), jnp.float32)
    return kern(fg, idx, rmax, cmax, csum, maskf, zeros)


def _stage3_body(mem_ref, inc_ref, out_ref):
    um = mem_ref[...] + inc_ref[0] + inc_ref[1]
    nrm = jnp.sqrt(jnp.sum(um * um, axis=1, keepdims=True))
    out_ref[...] = um / jnp.maximum(nrm, 1e-12)


def _stage3(memory, inc):
    M, C = memory.shape
    return pl.pallas_call(
        _stage3_body,
        out_shape=jax.ShapeDtypeStruct((M, C), jnp.float32),
    )(memory, inc)


def kernel(feature, memory, train, mask):
    B, C, D = feature.shape
    M = memory.shape[0]
    ft = jnp.transpose(feature, (2, 0, 1))
    maskf = mask.astype(jnp.float32) * jnp.asarray(train, jnp.float32)
    gct, fg, score = _stage1(ft, memory, 32)
    gc = jnp.transpose(gct, (1, 2, 0))
    cmax, csum, idx2, rmax2 = _stage15(score)
    inc = _stage2_sc(fg, idx2.reshape(B), rmax2.reshape(B),
                     cmax.reshape(M), csum.reshape(M), maskf)
    upd = _stage3(memory, inc)
    return gc, upd
